# Initial kernel scaffold; baseline (speedup 1.0000x reference)
import jax
import jax.numpy as jnp
from jax.experimental import pallas as pl


def kernel(x, e_outputs, src_mask, trg_mask, w_gate, w_noise, Wq, Wk, Wv, Wo, W1, b1, W2, b2, ln1_g, ln1_b, ln2_g, ln2_b):
    raise NotImplementedError("write your pallas kernel here")


# speedup vs baseline: 1.7068x; 1.7068x over previous
import functools

import jax
import jax.numpy as jnp
import numpy as np
from jax.experimental import pallas as pl
from jax.experimental.pallas import tpu as pltpu

B, S, D, H, E, K, DFF = 1, 2048, 768, 12, 8, 2, 3072
DH = D // H
NEXP = B * K

BS = 512
BQ = 512
BK = 512


def _erf(z):
    a1, a2, a3, a4, a5, p = (0.254829592, -0.284496736, 1.421413741,
                             -1.453152027, 1.061405429, 0.3275911)
    az = jnp.abs(z)
    t = 1.0 / (1.0 + p * az)
    poly = ((((a5 * t + a4) * t + a3) * t + a2) * t + a1) * t
    y = 1.0 - poly * jnp.exp(-az * az)
    return jnp.where(z >= 0.0, y, -y)


def _ncdf(z):
    return 0.5 * (1.0 + _erf(z * np.float32(1.0 / np.sqrt(2.0))))


def _gating_kernel(x_ref, wg_ref, wn_ref, noise_ref,
                   ei_ref, ng_ref, loss_ref):
    xsum = jnp.sum(x_ref[...], axis=0, keepdims=True)
    clean = jnp.dot(xsum, wg_ref[...],
                    preferred_element_type=jnp.float32)
    raw = jnp.dot(xsum, wn_ref[...],
                  preferred_element_type=jnp.float32)
    sp = jnp.maximum(raw, 0.0) + jnp.log(1.0 + jnp.exp(-jnp.abs(raw)))
    stddev = sp + 0.01
    noisy = clean + noise_ref[...] * stddev

    iota = jax.lax.broadcasted_iota(jnp.int32, (1, E), 1)
    neg = jnp.float32(-jnp.inf)

    work = noisy
    v0 = jnp.max(work)
    i0 = jnp.min(jnp.where(work == v0, iota, E))
    work = jnp.where(iota == i0, neg, work)
    v1 = jnp.max(work)
    i1 = jnp.min(jnp.where(work == v1, iota, E))
    work = jnp.where(iota == i1, neg, work)
    v2 = jnp.max(work)

    mx = jnp.maximum(v0, v1)
    e0 = jnp.exp(v0 - mx)
    e1 = jnp.exp(v1 - mx)
    z = e0 + e1
    g0 = e0 / z
    g1 = e1 / z

    th_in = v2
    th_out = v1
    is_in = noisy > th_in
    p_in = _ncdf((clean - th_in) / stddev)
    p_out = _ncdf((clean - th_out) / stddev)
    load = jnp.where(is_in, p_in, p_out)
    gates = (jnp.where(iota == i0, g0, 0.0)
             + jnp.where(iota == i1, g1, 0.0))

    def cv(v):
        mu = jnp.mean(v)
        var = jnp.sum((v - mu) ** 2) / (E - 1)
        return var / (mu * mu + 1e-10)

    loss = (cv(gates) + cv(load)) * 0.01

    lo_first = i0 < i1
    ei_lo = jnp.where(lo_first, i0, i1)
    ei_hi = jnp.where(lo_first, i1, i0)
    ng_lo = jnp.where(lo_first, g0, g1)
    ng_hi = jnp.where(lo_first, g1, g0)

    k_iota = jax.lax.broadcasted_iota(jnp.int32, (1, K), 1)
    ei_ref[...] = jnp.where(k_iota == 0, ei_lo, ei_hi).astype(jnp.int32)
    ng_ref[...] = jnp.where(k_iota == 0, ng_lo, ng_hi)
    loss_ref[...] = jnp.full((1, 1), loss, jnp.float32)


def _gating(x2d, w_gate, w_noise, noise):
    return pl.pallas_call(
        _gating_kernel,
        out_shape=(jax.ShapeDtypeStruct((1, K), jnp.int32),
                   jax.ShapeDtypeStruct((1, K), jnp.float32),
                   jax.ShapeDtypeStruct((1, 1), jnp.float32)),
    )(x2d, w_gate, w_noise, noise)


def _qkv_kernel(ei_ref, x_ref, wq_ref, wk_ref, wv_ref, q_ref, k_ref, v_ref):
    del ei_ref
    xb = x_ref[0]
    q_ref[0] = jnp.dot(xb, wq_ref[0], preferred_element_type=jnp.float32)
    k_ref[0] = jnp.dot(xb, wk_ref[0], preferred_element_type=jnp.float32)
    v_ref[0] = jnp.dot(xb, wv_ref[0], preferred_element_type=jnp.float32)


def _qkv(ei, x3, Wq, Wk, Wv):
    w_spec = pl.BlockSpec((1, D, D), lambda n, s, ei: (ei[n], 0, 0))
    o_spec = pl.BlockSpec((1, BS, D), lambda n, s, ei: (n, s, 0))
    return pl.pallas_call(
        _qkv_kernel,
        grid_spec=pltpu.PrefetchScalarGridSpec(
            num_scalar_prefetch=1,
            grid=(NEXP, S // BS),
            in_specs=[pl.BlockSpec((1, BS, D), lambda n, s, ei: (0, s, 0)),
                      w_spec, w_spec, w_spec],
            out_specs=[o_spec, o_spec, o_spec],
        ),
        out_shape=[jax.ShapeDtypeStruct((NEXP, S, D), jnp.float32)] * 3,
        compiler_params=pltpu.CompilerParams(
            dimension_semantics=("arbitrary", "arbitrary")),
    )(ei, x3, Wq, Wk, Wv)


def _attn_kernel(q_ref, k_ref, v_ref, o_ref):
    i = pl.program_id(2)
    scale = np.float32(1.0 / np.sqrt(DH))
    q = q_ref[0, 0] * scale

    row = jax.lax.broadcasted_iota(jnp.int32, (BQ, BK), 0) + i * BQ

    def body(j, carry):
        acc, m, l = carry
        kb = k_ref[0, 0, pl.ds(j * BK, BK), :]
        vb = v_ref[0, 0, pl.ds(j * BK, BK), :]
        s = jax.lax.dot_general(q, kb, (((1,), (1,)), ((), ())),
                                preferred_element_type=jnp.float32)
        col = jax.lax.broadcasted_iota(jnp.int32, (BQ, BK), 1) + j * BK
        s = jnp.where(row >= col, s, -1e9)
        m_new = jnp.maximum(m, jnp.max(s, axis=1, keepdims=True))
        p = jnp.exp(s - m_new)
        alpha = jnp.exp(m - m_new)
        l = l * alpha + jnp.sum(p, axis=1, keepdims=True)
        acc = acc * alpha + jnp.dot(p, vb,
                                    preferred_element_type=jnp.float32)
        return acc, m_new, l

    acc = jnp.zeros((BQ, DH), jnp.float32)
    m = jnp.full((BQ, 1), -jnp.inf, jnp.float32)
    l = jnp.zeros((BQ, 1), jnp.float32)
    acc, m, l = jax.lax.fori_loop(0, i + 1, body, (acc, m, l))
    o_ref[0, 0] = acc / l


def _attention(qh, kh, vh):
    kv_spec = pl.BlockSpec((1, 1, S, DH), lambda n, h, i: (n, h, 0, 0))
    q_spec = pl.BlockSpec((1, 1, BQ, DH), lambda n, h, i: (n, h, i, 0))
    return pl.pallas_call(
        _attn_kernel,
        grid=(NEXP, H, S // BQ),
        in_specs=[q_spec, kv_spec, kv_spec],
        out_specs=q_spec,
        out_shape=jax.ShapeDtypeStruct((NEXP, H, S, DH), jnp.float32),
        compiler_params=pltpu.CompilerParams(
            dimension_semantics=("arbitrary", "arbitrary", "arbitrary")),
    )(qh, kh, vh)


def _ln_rows(z, g, b):
    mu = jnp.mean(z, axis=-1, keepdims=True)
    var = jnp.mean((z - mu) ** 2, axis=-1, keepdims=True)
    return (z - mu) * jax.lax.rsqrt(var + 1e-5) * g + b


def _oproj_kernel(ei_ref, o_ref, x_ref, wo_ref, g_ref, b_ref, h1_ref):
    del ei_ref
    o = jnp.dot(o_ref[0], wo_ref[0], preferred_element_type=jnp.float32)
    z = x_ref[0] + o
    h1_ref[0] = _ln_rows(z, g_ref[0], b_ref[0])


def _oproj_ln1(ei, o3, x3, Wo, ln1_g, ln1_b):
    ln1_g = ln1_g.reshape(E, 1, D)
    ln1_b = ln1_b.reshape(E, 1, D)
    return pl.pallas_call(
        _oproj_kernel,
        grid_spec=pltpu.PrefetchScalarGridSpec(
            num_scalar_prefetch=1,
            grid=(NEXP, S // BS),
            in_specs=[
                pl.BlockSpec((1, BS, D), lambda n, s, ei: (n, s, 0)),
                pl.BlockSpec((1, BS, D), lambda n, s, ei: (0, s, 0)),
                pl.BlockSpec((1, D, D), lambda n, s, ei: (ei[n], 0, 0)),
                pl.BlockSpec((1, 1, D), lambda n, s, ei: (ei[n], 0, 0)),
                pl.BlockSpec((1, 1, D), lambda n, s, ei: (ei[n], 0, 0)),
            ],
            out_specs=pl.BlockSpec((1, BS, D), lambda n, s, ei: (n, s, 0)),
        ),
        out_shape=jax.ShapeDtypeStruct((NEXP, S, D), jnp.float32),
        compiler_params=pltpu.CompilerParams(
            dimension_semantics=("arbitrary", "arbitrary")),
    )(ei, o3, x3, Wo, ln1_g, ln1_b)


def _ffn_kernel(ei_ref, h1_ref, w1_ref, b1_ref, w2_ref, b2_ref,
                g2_ref, bt2_ref, y_ref):
    del ei_ref
    h1 = h1_ref[0]
    ff = jnp.dot(h1, w1_ref[0], preferred_element_type=jnp.float32)
    ff = jnp.maximum(ff + b1_ref[0], 0.0)
    ff = jnp.dot(ff, w2_ref[0], preferred_element_type=jnp.float32)
    ff = ff + b2_ref[0]
    y_ref[0] = _ln_rows(h1 + ff, g2_ref[0], bt2_ref[0])


def _ffn_ln2(ei, h1, W1, b1, W2, b2, ln2_g, ln2_b):
    b1 = b1.reshape(E, 1, DFF)
    b2 = b2.reshape(E, 1, D)
    ln2_g = ln2_g.reshape(E, 1, D)
    ln2_b = ln2_b.reshape(E, 1, D)
    vec_spec = pl.BlockSpec((1, 1, D), lambda n, s, ei: (ei[n], 0, 0))
    return pl.pallas_call(
        _ffn_kernel,
        grid_spec=pltpu.PrefetchScalarGridSpec(
            num_scalar_prefetch=1,
            grid=(NEXP, S // BS),
            in_specs=[
                pl.BlockSpec((1, BS, D), lambda n, s, ei: (n, s, 0)),
                pl.BlockSpec((1, D, DFF), lambda n, s, ei: (ei[n], 0, 0)),
                pl.BlockSpec((1, 1, DFF), lambda n, s, ei: (ei[n], 0, 0)),
                pl.BlockSpec((1, DFF, D), lambda n, s, ei: (ei[n], 0, 0)),
                vec_spec,
                vec_spec,
                vec_spec,
            ],
            out_specs=pl.BlockSpec((1, BS, D), lambda n, s, ei: (n, s, 0)),
        ),
        out_shape=jax.ShapeDtypeStruct((NEXP, S, D), jnp.float32),
        compiler_params=pltpu.CompilerParams(
            dimension_semantics=("arbitrary", "arbitrary")),
    )(ei, h1, W1, b1, W2, b2, ln2_g, ln2_b)


def _combine_kernel(y_ref, ng_ref, out_ref):
    acc = y_ref[0] * ng_ref[0, 0] + y_ref[1] * ng_ref[0, 1]
    eps = jnp.float32(np.finfo(np.float64).eps)
    out_ref[0] = jnp.where(acc == 0.0, eps, acc)


def _combine(y, ng):
    return pl.pallas_call(
        _combine_kernel,
        grid=(S // BS,),
        in_specs=[pl.BlockSpec((NEXP, BS, D), lambda s: (0, s, 0)),
                  pl.BlockSpec((1, K), lambda s: (0, 0))],
        out_specs=pl.BlockSpec((1, BS, D), lambda s: (0, s, 0)),
        out_shape=jax.ShapeDtypeStruct((B, S, D), jnp.float32),
        compiler_params=pltpu.CompilerParams(
            dimension_semantics=("arbitrary",)),
    )(y, ng)


def kernel(x, e_outputs, src_mask, trg_mask, w_gate, w_noise, Wq, Wk, Wv, Wo,
           W1, b1, W2, b2, ln1_g, ln1_b, ln2_g, ln2_b):
    del e_outputs, src_mask, trg_mask
    x2d = x[0]
    noise = jax.random.normal(jax.random.key(42), (B, E), jnp.float32)
    ei2d, ng, loss2d = _gating(x2d, w_gate, w_noise, noise)
    ei = ei2d.reshape(K)

    x3 = x
    q, k, v = _qkv(ei, x3, Wq, Wk, Wv)
    qh = q.reshape(NEXP, S, H, DH).transpose(0, 2, 1, 3)
    kh = k.reshape(NEXP, S, H, DH).transpose(0, 2, 1, 3)
    vh = v.reshape(NEXP, S, H, DH).transpose(0, 2, 1, 3)
    oh = _attention(qh, kh, vh)
    o3 = oh.transpose(0, 2, 1, 3).reshape(NEXP, S, D)
    h1 = _oproj_ln1(ei, o3, x3, Wo, ln1_g, ln1_b)
    y = _ffn_ln2(ei, h1, W1, b1, W2, b2, ln2_g, ln2_b)
    combined = _combine(y, ng)
    return combined, loss2d.reshape(())


# speedup vs baseline: 1.7923x; 1.0501x over previous
import functools

import jax
import jax.numpy as jnp
import numpy as np
from jax.experimental import pallas as pl
from jax.experimental.pallas import tpu as pltpu

B, S, D, H, E, K, DFF = 1, 2048, 768, 12, 8, 2, 3072
DH = D // H
NEXP = B * K

BS = 512
BQ = 512
BK = 512


def _erf(z):
    a1, a2, a3, a4, a5, p = (0.254829592, -0.284496736, 1.421413741,
                             -1.453152027, 1.061405429, 0.3275911)
    az = jnp.abs(z)
    t = 1.0 / (1.0 + p * az)
    poly = ((((a5 * t + a4) * t + a3) * t + a2) * t + a1) * t
    y = 1.0 - poly * jnp.exp(-az * az)
    return jnp.where(z >= 0.0, y, -y)


def _ncdf(z):
    return 0.5 * (1.0 + _erf(z * np.float32(1.0 / np.sqrt(2.0))))


def _gating_kernel(x_ref, wg_ref, wn_ref, noise_ref,
                   ei_ref, ng_ref, loss_ref):
    xsum = jnp.sum(x_ref[...], axis=0, keepdims=True)
    clean = jnp.dot(xsum, wg_ref[...],
                    preferred_element_type=jnp.float32)
    raw = jnp.dot(xsum, wn_ref[...],
                  preferred_element_type=jnp.float32)
    sp = jnp.maximum(raw, 0.0) + jnp.log(1.0 + jnp.exp(-jnp.abs(raw)))
    stddev = sp + 0.01
    noisy = clean + noise_ref[...] * stddev

    iota = jax.lax.broadcasted_iota(jnp.int32, (1, E), 1)
    neg = jnp.float32(-jnp.inf)

    work = noisy
    v0 = jnp.max(work)
    i0 = jnp.min(jnp.where(work == v0, iota, E))
    work = jnp.where(iota == i0, neg, work)
    v1 = jnp.max(work)
    i1 = jnp.min(jnp.where(work == v1, iota, E))
    work = jnp.where(iota == i1, neg, work)
    v2 = jnp.max(work)

    mx = jnp.maximum(v0, v1)
    e0 = jnp.exp(v0 - mx)
    e1 = jnp.exp(v1 - mx)
    z = e0 + e1
    g0 = e0 / z
    g1 = e1 / z

    th_in = v2
    th_out = v1
    is_in = noisy > th_in
    p_in = _ncdf((clean - th_in) / stddev)
    p_out = _ncdf((clean - th_out) / stddev)
    load = jnp.where(is_in, p_in, p_out)
    gates = (jnp.where(iota == i0, g0, 0.0)
             + jnp.where(iota == i1, g1, 0.0))

    def cv(v):
        mu = jnp.mean(v)
        var = jnp.sum((v - mu) ** 2) / (E - 1)
        return var / (mu * mu + 1e-10)

    loss = (cv(gates) + cv(load)) * 0.01

    lo_first = i0 < i1
    ei_lo = jnp.where(lo_first, i0, i1)
    ei_hi = jnp.where(lo_first, i1, i0)
    ng_lo = jnp.where(lo_first, g0, g1)
    ng_hi = jnp.where(lo_first, g1, g0)

    k_iota = jax.lax.broadcasted_iota(jnp.int32, (1, K), 1)
    ei_ref[...] = jnp.where(k_iota == 0, ei_lo, ei_hi).astype(jnp.int32)
    ng_ref[...] = jnp.where(k_iota == 0, ng_lo, ng_hi)
    loss_ref[...] = jnp.full((1, 1), loss, jnp.float32)


def _gating(x2d, w_gate, w_noise, noise):
    return pl.pallas_call(
        _gating_kernel,
        out_shape=(jax.ShapeDtypeStruct((1, K), jnp.int32),
                   jax.ShapeDtypeStruct((1, K), jnp.float32),
                   jax.ShapeDtypeStruct((1, 1), jnp.float32)),
    )(x2d, w_gate, w_noise, noise)


def _qkv_kernel(ei_ref, x_ref, wq_ref, wk_ref, wv_ref, q_ref, k_ref, v_ref):
    del ei_ref
    xb = x_ref[0].astype(jnp.bfloat16)
    q_ref[0] = jnp.dot(xb, wq_ref[0].astype(jnp.bfloat16),
                       preferred_element_type=jnp.float32).astype(jnp.bfloat16)
    k_ref[0] = jnp.dot(xb, wk_ref[0].astype(jnp.bfloat16),
                       preferred_element_type=jnp.float32).astype(jnp.bfloat16)
    v_ref[0] = jnp.dot(xb, wv_ref[0].astype(jnp.bfloat16),
                       preferred_element_type=jnp.float32).astype(jnp.bfloat16)


def _qkv(ei, x3, Wq, Wk, Wv):
    w_spec = pl.BlockSpec((1, D, D), lambda n, s, ei: (ei[n], 0, 0))
    o_spec = pl.BlockSpec((1, BS, D), lambda n, s, ei: (n, s, 0))
    return pl.pallas_call(
        _qkv_kernel,
        grid_spec=pltpu.PrefetchScalarGridSpec(
            num_scalar_prefetch=1,
            grid=(NEXP, S // BS),
            in_specs=[pl.BlockSpec((1, BS, D), lambda n, s, ei: (0, s, 0)),
                      w_spec, w_spec, w_spec],
            out_specs=[o_spec, o_spec, o_spec],
        ),
        out_shape=[jax.ShapeDtypeStruct((NEXP, S, D), jnp.bfloat16)] * 3,
        compiler_params=pltpu.CompilerParams(
            dimension_semantics=("arbitrary", "arbitrary")),
    )(ei, x3, Wq, Wk, Wv)


def _attn_kernel(q_ref, k_ref, v_ref, o_ref):
    i = pl.program_id(2)
    scale = np.float32(1.0 / np.sqrt(DH))
    q = q_ref[0, 0]

    row = jax.lax.broadcasted_iota(jnp.int32, (BQ, BK), 0) + i * BQ

    def body(j, carry):
        acc, m, l = carry
        kb = k_ref[0, 0, pl.ds(j * BK, BK), :]
        vb = v_ref[0, 0, pl.ds(j * BK, BK), :]
        s = jax.lax.dot_general(q, kb, (((1,), (1,)), ((), ())),
                                preferred_element_type=jnp.float32) * scale
        col = jax.lax.broadcasted_iota(jnp.int32, (BQ, BK), 1) + j * BK
        s = jnp.where(row >= col, s, -1e9)
        m_new = jnp.maximum(m, jnp.max(s, axis=1, keepdims=True))
        p = jnp.exp(s - m_new)
        alpha = jnp.exp(m - m_new)
        l = l * alpha + jnp.sum(p, axis=1, keepdims=True)
        acc = acc * alpha + jnp.dot(p.astype(jnp.bfloat16), vb,
                                    preferred_element_type=jnp.float32)
        return acc, m_new, l

    acc = jnp.zeros((BQ, DH), jnp.float32)
    m = jnp.full((BQ, 1), -jnp.inf, jnp.float32)
    l = jnp.zeros((BQ, 1), jnp.float32)
    acc, m, l = jax.lax.fori_loop(0, i + 1, body, (acc, m, l))
    o_ref[0, 0] = (acc / l).astype(jnp.bfloat16)


def _attention(qh, kh, vh):
    kv_spec = pl.BlockSpec((1, 1, S, DH), lambda n, h, i: (n, h, 0, 0))
    q_spec = pl.BlockSpec((1, 1, BQ, DH), lambda n, h, i: (n, h, i, 0))
    return pl.pallas_call(
        _attn_kernel,
        grid=(NEXP, H, S // BQ),
        in_specs=[q_spec, kv_spec, kv_spec],
        out_specs=q_spec,
        out_shape=jax.ShapeDtypeStruct((NEXP, H, S, DH), jnp.bfloat16),
        compiler_params=pltpu.CompilerParams(
            dimension_semantics=("arbitrary", "arbitrary", "arbitrary")),
    )(qh, kh, vh)


def _ln_rows(z, g, b):
    mu = jnp.mean(z, axis=-1, keepdims=True)
    var = jnp.mean((z - mu) ** 2, axis=-1, keepdims=True)
    return (z - mu) * jax.lax.rsqrt(var + 1e-5) * g + b


def _oproj_kernel(ei_ref, o_ref, x_ref, wo_ref, g_ref, b_ref, h1_ref):
    del ei_ref
    o = jnp.dot(o_ref[0], wo_ref[0].astype(jnp.bfloat16),
                preferred_element_type=jnp.float32)
    z = x_ref[0] + o
    h1_ref[0] = _ln_rows(z, g_ref[0], b_ref[0])


def _oproj_ln1(ei, o3, x3, Wo, ln1_g, ln1_b):
    ln1_g = ln1_g.reshape(E, 1, D)
    ln1_b = ln1_b.reshape(E, 1, D)
    return pl.pallas_call(
        _oproj_kernel,
        grid_spec=pltpu.PrefetchScalarGridSpec(
            num_scalar_prefetch=1,
            grid=(NEXP, S // BS),
            in_specs=[
                pl.BlockSpec((1, BS, D), lambda n, s, ei: (n, s, 0)),
                pl.BlockSpec((1, BS, D), lambda n, s, ei: (0, s, 0)),
                pl.BlockSpec((1, D, D), lambda n, s, ei: (ei[n], 0, 0)),
                pl.BlockSpec((1, 1, D), lambda n, s, ei: (ei[n], 0, 0)),
                pl.BlockSpec((1, 1, D), lambda n, s, ei: (ei[n], 0, 0)),
            ],
            out_specs=pl.BlockSpec((1, BS, D), lambda n, s, ei: (n, s, 0)),
        ),
        out_shape=jax.ShapeDtypeStruct((NEXP, S, D), jnp.float32),
        compiler_params=pltpu.CompilerParams(
            dimension_semantics=("arbitrary", "arbitrary")),
    )(ei, o3, x3, Wo, ln1_g, ln1_b)


def _ffn_kernel(ei_ref, h1_ref, w1_ref, b1_ref, w2_ref, b2_ref,
                g2_ref, bt2_ref, y_ref):
    del ei_ref
    h1 = h1_ref[0]
    ff = jnp.dot(h1.astype(jnp.bfloat16), w1_ref[0].astype(jnp.bfloat16),
                 preferred_element_type=jnp.float32)
    ff = jnp.maximum(ff + b1_ref[0], 0.0)
    ff = jnp.dot(ff.astype(jnp.bfloat16), w2_ref[0].astype(jnp.bfloat16),
                 preferred_element_type=jnp.float32)
    ff = ff + b2_ref[0]
    y_ref[0] = _ln_rows(h1 + ff, g2_ref[0], bt2_ref[0])


def _ffn_ln2(ei, h1, W1, b1, W2, b2, ln2_g, ln2_b):
    b1 = b1.reshape(E, 1, DFF)
    b2 = b2.reshape(E, 1, D)
    ln2_g = ln2_g.reshape(E, 1, D)
    ln2_b = ln2_b.reshape(E, 1, D)
    vec_spec = pl.BlockSpec((1, 1, D), lambda n, s, ei: (ei[n], 0, 0))
    return pl.pallas_call(
        _ffn_kernel,
        grid_spec=pltpu.PrefetchScalarGridSpec(
            num_scalar_prefetch=1,
            grid=(NEXP, S // BS),
            in_specs=[
                pl.BlockSpec((1, BS, D), lambda n, s, ei: (n, s, 0)),
                pl.BlockSpec((1, D, DFF), lambda n, s, ei: (ei[n], 0, 0)),
                pl.BlockSpec((1, 1, DFF), lambda n, s, ei: (ei[n], 0, 0)),
                pl.BlockSpec((1, DFF, D), lambda n, s, ei: (ei[n], 0, 0)),
                vec_spec,
                vec_spec,
                vec_spec,
            ],
            out_specs=pl.BlockSpec((1, BS, D), lambda n, s, ei: (n, s, 0)),
        ),
        out_shape=jax.ShapeDtypeStruct((NEXP, S, D), jnp.float32),
        compiler_params=pltpu.CompilerParams(
            dimension_semantics=("arbitrary", "arbitrary")),
    )(ei, h1, W1, b1, W2, b2, ln2_g, ln2_b)


def _combine_kernel(y_ref, ng_ref, out_ref):
    acc = y_ref[0] * ng_ref[0, 0] + y_ref[1] * ng_ref[0, 1]
    eps = jnp.float32(np.finfo(np.float64).eps)
    out_ref[0] = jnp.where(acc == 0.0, eps, acc)


def _combine(y, ng):
    return pl.pallas_call(
        _combine_kernel,
        grid=(S // BS,),
        in_specs=[pl.BlockSpec((NEXP, BS, D), lambda s: (0, s, 0)),
                  pl.BlockSpec((1, K), lambda s: (0, 0))],
        out_specs=pl.BlockSpec((1, BS, D), lambda s: (0, s, 0)),
        out_shape=jax.ShapeDtypeStruct((B, S, D), jnp.float32),
        compiler_params=pltpu.CompilerParams(
            dimension_semantics=("arbitrary",)),
    )(y, ng)


def kernel(x, e_outputs, src_mask, trg_mask, w_gate, w_noise, Wq, Wk, Wv, Wo,
           W1, b1, W2, b2, ln1_g, ln1_b, ln2_g, ln2_b):
    del e_outputs, src_mask, trg_mask
    x2d = x[0]
    noise = jax.random.normal(jax.random.key(42), (B, E), jnp.float32)
    ei2d, ng, loss2d = _gating(x2d, w_gate, w_noise, noise)
    ei = ei2d.reshape(K)

    x3 = x
    q, k, v = _qkv(ei, x3, Wq, Wk, Wv)
    qh = q.reshape(NEXP, S, H, DH).transpose(0, 2, 1, 3)
    kh = k.reshape(NEXP, S, H, DH).transpose(0, 2, 1, 3)
    vh = v.reshape(NEXP, S, H, DH).transpose(0, 2, 1, 3)
    oh = _attention(qh, kh, vh)
    o3 = oh.transpose(0, 2, 1, 3).reshape(NEXP, S, D)
    h1 = _oproj_ln1(ei, o3, x3, Wo, ln1_g, ln1_b)
    y = _ffn_ln2(ei, h1, W1, b1, W2, b2, ln2_g, ln2_b)
    combined = _combine(y, ng)
    return combined, loss2d.reshape(())
